# baseline (speedup 1.0000x reference)
import functools

import jax
import jax.numpy as jnp
from jax import lax
from jax.experimental import pallas as pl
from jax.experimental.pallas import tpu as pltpu
from jax.experimental.pallas import tpu_sc as plsc

_N = 10000
_E = 320000
_D = 128
_H = 64
_A = 0.5

_NC = 2
_NS = 16
_NW = _NC * _NS
_CH = 128
_NCHUNK = 80
_EPW = _CH * _NCHUNK
_EPAD = _NW * _EPW
_NP = 10240
_RPT = _NP // _NS

_F32 = jnp.float32


def _mesh():
    return plsc.VectorSubcoreMesh(
        core_axis_name="c", subcore_axis_name="s",
        num_cores=_NC, num_subcores=_NS)


class _GatherJob:
    def __init__(self, table, scaled):
        self.table = table
        self.scaled = scaled
        self.width = table.shape[1]


_SLAB = 8
_NSLAB = _NCHUNK // _SLAB


def _multi_pass(jobs, degcnt, src_r, dst_r, ew_r):
    need_w = degcnt or any(j.scaled for j in jobs)
    out_type = [jax.ShapeDtypeStruct((_NC, _NP, j.width), _F32) for j in jobs]
    if degcnt:
        out_type.append(jax.ShapeDtypeStruct((_NC, _NP, 16), _F32))
    scratch = []
    for _ in range(2):
        scratch.append(pltpu.VMEM((_SLAB, _CH), jnp.int32))
        scratch.append(pltpu.VMEM((_SLAB, _CH), jnp.int32))
        scratch.append(pltpu.VMEM((_SLAB, _CH), _F32))
        scratch.append(pltpu.SemaphoreType.DMA)
    for j in jobs:
        scratch.append(pltpu.VMEM((_CH, j.width), _F32))
        scratch.append(pltpu.VMEM((_CH, j.width), _F32))
        scratch.append(pltpu.SemaphoreType.DMA)
        scratch.append(pltpu.SemaphoreType.DMA)
        scratch.append(pltpu.VMEM_SHARED((_NP, j.width), _F32))
    if degcnt:
        scratch.append(pltpu.VMEM((_CH, 16), _F32))
        scratch.append(pltpu.VMEM_SHARED((_NP, 16), _F32))

    @functools.partial(
        pl.kernel,
        out_type=tuple(out_type),
        mesh=_mesh(),
        compiler_params=pltpu.CompilerParams(use_tc_tiling_on_sc=False),
        scratch_types=scratch)
    def k(*refs):
        ntab = len(jobs)
        nout = ntab + (1 if degcnt else 0)
        tables = refs[:ntab]
        src_hbm, dst_hbm, ew_hbm = refs[ntab:ntab + 3]
        outs = refs[ntab + 3:ntab + 3 + nout]
        sc = list(refs[ntab + 3 + nout:])
        src_sl = (sc[0], sc[4])
        dst_sl = (sc[1], sc[5])
        w_sl = (sc[2], sc[6])
        rsem = (sc[3], sc[7])
        jb = []
        for idx in range(ntab):
            jb.append(sc[8 + 5 * idx: 8 + 5 * (idx + 1)])
        if degcnt:
            rows16, acc16 = sc[8 + 5 * ntab:]

        c = lax.axis_index("c")
        s = lax.axis_index("s")
        wid = s * _NC + c
        r0 = s * _RPT

        def zero_acc(buf, accref, width):
            zv = jnp.zeros((16,), _F32)

            def zrow(e, zcarry):
                for cg in range(width // 16):
                    buf[e, pl.ds(cg * 16, 16)] = zv
                return zcarry
            lax.fori_loop(0, _CH, zrow, 0)
            for t in range(_RPT // _CH):
                pltpu.sync_copy(buf, accref.at[pl.ds(r0 + t * _CH, _CH)])

        for idx, j in enumerate(jobs):
            zero_acc(jb[idx][0], jb[idx][4], j.width)
        if degcnt:
            zero_acc(rows16, acc16, 16)
        plsc.subcore_barrier()

        def slab_copies(sl, p):
            rows = pl.ds(sl * _SLAB, _SLAB)
            cps = [(src_hbm.at[wid, rows], src_sl[p]),
                   (dst_hbm.at[wid, rows], dst_sl[p])]
            if need_w:
                cps.append((ew_hbm.at[wid, rows], w_sl[p]))
            return cps

        def slab_fetch(sl, p):
            for a, b in slab_copies(sl, p):
                pltpu.async_copy(a, b, rsem[p])

        def slab_wait(sl, p):
            for a, b in slab_copies(sl, p):
                pltpu.make_async_copy(a, b, rsem[p]).wait()

        iota = lax.iota(jnp.int32, 16)
        ones = jnp.ones((16,), _F32)
        zero = jnp.zeros((16,), _F32)

        def scale_rows(wref, brow, buf, width):
            def grp(g, gcarry):
                v16 = wref[brow, pl.ds(g * 16, 16)]
                for jj in range(16):
                    wb = jnp.full((16,), v16[jj], _F32)
                    e = g * 16 + jj
                    for cg in range(width // 16):
                        sl = pl.ds(cg * 16, 16)
                        buf[e, sl] = buf[e, sl] * wb
                return gcarry
            lax.fori_loop(0, _CH // 16, grp, 0)

        def degcnt_rows(wref, brow):
            def grp(g, gcarry):
                v16 = wref[brow, pl.ds(g * 16, 16)]
                for jj in range(16):
                    wb = jnp.full((16,), v16[jj], _F32)
                    rows16[g * 16 + jj, :] = jnp.where(
                        iota == 0, wb, jnp.where(iota == 1, ones, zero))
                return gcarry
            lax.fori_loop(0, _CH // 16, grp, 0)

        slab_fetch(0, 0)
        slab_wait(0, 0)
        slab_fetch(1, 1)
        for idx in range(ntab):
            pltpu.async_copy(tables[idx].at[src_sl[0].at[0]], jb[idx][0],
                             jb[idx][2])

        def outer(sp, carry):
            for p in range(2):
                sl = sp * 2 + p
                for b in range(_SLAB):
                    i = sl * _SLAB + b
                    if b == _SLAB - 1:
                        @pl.when(sl + 1 < _NSLAB)
                        def _():
                            slab_wait(sl + 1, 1 - p)
                    nrow = b + 1 if b < _SLAB - 1 else 0
                    npar = p if b < _SLAB - 1 else 1 - p

                    @pl.when(i + 1 < _NCHUNK)
                    def _():
                        for idx in range(ntab):
                            pltpu.async_copy(
                                tables[idx].at[src_sl[npar].at[nrow]],
                                jb[idx][(b + 1) % 2],
                                jb[idx][2 + (b + 1) % 2])
                    for idx, j in enumerate(jobs):
                        buf = jb[idx][b % 2]
                        pltpu.make_async_copy(
                            tables[idx].at[src_sl[p].at[b]], buf,
                            jb[idx][2 + b % 2]).wait()
                        if j.scaled:
                            scale_rows(w_sl[p], b, buf, j.width)
                        pltpu.sync_copy(buf, jb[idx][4].at[dst_sl[p].at[b]],
                                        add=True)
                    if degcnt:
                        degcnt_rows(w_sl[p], b)
                        pltpu.sync_copy(rows16, acc16.at[dst_sl[p].at[b]],
                                        add=True)

                @pl.when(sl + 2 < _NSLAB)
                def _():
                    slab_fetch(sl + 2, p)
            return carry
        lax.fori_loop(0, _NSLAB // 2, outer, 0)

        plsc.subcore_barrier()
        for idx in range(ntab):
            pltpu.sync_copy(jb[idx][4].at[pl.ds(r0, _RPT)],
                            outs[idx].at[c, pl.ds(r0, _RPT)])
        if degcnt:
            pltpu.sync_copy(acc16.at[pl.ds(r0, _RPT)],
                            outs[ntab].at[c, pl.ds(r0, _RPT)])

    return k(*[j.table for j in jobs], src_r, dst_r, ew_r)


_TC_PARAMS = pltpu.CompilerParams(vmem_limit_bytes=100 * 1024 * 1024)
_BN = 2000
_GRID = _N // _BN


def _dot(a, b):
    return jnp.dot(a, b, preferred_element_type=_F32,
                   precision=lax.Precision.HIGHEST)


def _bs_acc(width):
    return pl.BlockSpec((_NC, _BN, width), lambda i: (0, i, 0))


def _bs_rows(width):
    return pl.BlockSpec((_BN, width), lambda i: (i, 0))


def _bs_full2(shape):
    return pl.BlockSpec(shape, lambda i: (0, 0))


def _bs_full1(shape):
    return pl.BlockSpec(shape, lambda i: (0,))


def _tc1(acc_a, x, W1):
    def body(acc_ref, x_ref, w1_ref, dis_ref, cnt_ref, h1_ref, h1p_ref):
        a = acc_ref[0] + acc_ref[1]
        deg = a[:, 0:1] + 1.0
        dis = lax.rsqrt(deg)
        h1 = _dot(x_ref[...], w1_ref[...])
        dis_ref[...] = dis
        cnt_ref[...] = a[:, 1:2]
        h1_ref[...] = h1
        h1p_ref[...] = h1 * dis
    return pl.pallas_call(
        body,
        grid=(_GRID,),
        in_specs=[_bs_acc(16), _bs_rows(_D), _bs_full2((_D, _H))],
        out_specs=(_bs_rows(1), _bs_rows(1), _bs_rows(_H), _bs_rows(_H)),
        compiler_params=_TC_PARAMS,
        out_shape=(jax.ShapeDtypeStruct((_N, 1), _F32),
                   jax.ShapeDtypeStruct((_N, 1), _F32),
                   jax.ShapeDtypeStruct((_N, _H), _F32),
                   jax.ShapeDtypeStruct((_N, _H), _F32)))(acc_a, x, W1)


def _tc2(acc_b, dis, h1, b1, W2):
    def body(acc_ref, dis_ref, h1_ref, b1_ref, w2_ref, h2_ref, h2p_ref):
        dis = dis_ref[...]
        s1 = acc_ref[0] + acc_ref[1]
        z1 = jnp.maximum(dis * s1 + (dis * dis) * h1_ref[...] + b1_ref[...], 0.0)
        h2 = _dot(z1, w2_ref[...])
        h2_ref[...] = h2
        h2p_ref[...] = h2 * dis
    return pl.pallas_call(
        body,
        grid=(_GRID,),
        in_specs=[_bs_acc(_H), _bs_rows(1), _bs_rows(_H), _bs_full1((_H,)),
                  _bs_full2((_H, _H))],
        out_specs=(_bs_rows(_H), _bs_rows(_H)),
        compiler_params=_TC_PARAMS,
        out_shape=(jax.ShapeDtypeStruct((_N, _H), _F32),
                   jax.ShapeDtypeStruct((_N, _H), _F32)))(acc_b, dis, h1, b1, W2)


def _tc3(acc_c, dis, h2, b2, x, Wa1, ba1, Wa2, ba2):
    def body(acc_ref, dis_ref, h2_ref, b2_ref, x_ref,
             wa1_ref, ba1_ref, wa2_ref, ba2_ref, z_ref, err_ref):
        dis = dis_ref[...]
        s2 = acc_ref[0] + acc_ref[1]
        z = jnp.maximum(dis * s2 + (dis * dis) * h2_ref[...] + b2_ref[...], 0.0)
        u = jnp.maximum(_dot(z, wa1_ref[...]) + ba1_ref[...], 0.0)
        x_hat = jax.nn.sigmoid(_dot(u, wa2_ref[...]) + ba2_ref[...])
        d = x_hat - x_ref[...]
        z_ref[...] = z
        err_ref[...] = jnp.sqrt(jnp.sum(d * d, axis=1, keepdims=True) + 1e-12)
    return pl.pallas_call(
        body,
        grid=(_GRID,),
        in_specs=[_bs_acc(_H), _bs_rows(1), _bs_rows(_H), _bs_full1((_H,)),
                  _bs_rows(_D), _bs_full2((_H, _H)), _bs_full1((_H,)),
                  _bs_full2((_H, _D)), _bs_full1((_D,))],
        out_specs=(_bs_rows(_H), _bs_rows(1)),
        compiler_params=_TC_PARAMS,
        out_shape=(jax.ShapeDtypeStruct((_N, _H), _F32),
                   jax.ShapeDtypeStruct((_N, 1), _F32)))(
            acc_c, dis, h2, b2, x, Wa1, ba1, Wa2, ba2)


def _tc4(acc_d1, acc_d2a, acc_d2b, cnt, attr_err, Wh1, bh1, Wh2, bh2):
    def body(d1_ref, d2a_ref, d2b_ref, cnt_ref, attr_ref,
             wh1_ref, bh1_ref, wh2_ref, bh2_ref, score_ref):
        inv = 1.0 / jnp.maximum(cnt_ref[...], 1.0)
        z_bar = (d1_ref[0] + d1_ref[1]) * inv
        m_xa = (d2a_ref[0] + d2a_ref[1]) * inv
        m_xb = (d2b_ref[0] + d2b_ref[1]) * inv
        v = jnp.maximum(_dot(z_bar, wh1_ref[...]) + bh1_ref[...], 0.0)
        x_homo = jax.nn.sigmoid(_dot(v, wh2_ref[...]) + bh2_ref[...])
        da = x_homo[:, :_H] - m_xa
        db = x_homo[:, _H:] - m_xb
        homo = jnp.sqrt(jnp.sum(da * da, axis=1) + jnp.sum(db * db, axis=1)
                        + 1e-12)
        score_ref[...] = (_A * attr_ref[:, 0]
                          + (1.0 - _A) * homo)[:, None]
    return pl.pallas_call(
        body,
        grid=(_GRID,),
        in_specs=[_bs_acc(_H), _bs_acc(_H), _bs_acc(_H), _bs_rows(1),
                  _bs_rows(1), _bs_full2((_H, _H)), _bs_full1((_H,)),
                  _bs_full2((_H, _D)), _bs_full1((_D,))],
        out_specs=_bs_rows(1),
        compiler_params=_TC_PARAMS,
        out_shape=jax.ShapeDtypeStruct((_N, 1), _F32))(
            acc_d1, acc_d2a, acc_d2b, cnt, attr_err, Wh1, bh1, Wh2, bh2)


def kernel(x, edge_index, edge_weight, W1, b1, W2, b2,
           Wa1, ba1, Wa2, ba2, Wh1, bh1, Wh2, bh2):
    src = edge_index[0]
    dst = edge_index[1]
    npad = _EPAD - _E
    src_p = jnp.concatenate([src, jnp.zeros((npad,), jnp.int32)])
    dst_p = jnp.concatenate(
        [dst, _N + (jnp.arange(npad, dtype=jnp.int32) % (_NP - _N))])
    ew_p = jnp.concatenate([edge_weight, jnp.zeros((npad,), _F32)])
    src_r = src_p.reshape(_NW, _NCHUNK, _CH)
    dst_r = dst_p.reshape(_NW, _NCHUNK, _CH)
    ew_r = ew_p.reshape(_NW, _NCHUNK, _CH)

    acc_d2a, acc_a = _multi_pass([_GatherJob(x[:, :_H], False)], True,
                                 src_r, dst_r, ew_r)
    dis, cnt, h1, h1p = _tc1(acc_a, x, W1)
    acc_b, acc_d2b = _multi_pass(
        [_GatherJob(h1p, True), _GatherJob(x[:, _H:], False)], False,
        src_r, dst_r, ew_r)
    h2, h2p = _tc2(acc_b, dis, h1, b1, W2)
    acc_c, = _multi_pass([_GatherJob(h2p, True)], False, src_r, dst_r, ew_r)
    z, attr_err = _tc3(acc_c, dis, h2, b2, x, Wa1, ba1, Wa2, ba2)
    acc_d1, = _multi_pass([_GatherJob(z, False)], False, src_r, dst_r, ew_r)
    score = _tc4(acc_d1, acc_d2a, acc_d2b, cnt, attr_err, Wh1, bh1, Wh2, bh2)
    return score[:, 0]
